# Initial kernel scaffold; baseline (speedup 1.0000x reference)
import jax
import jax.numpy as jnp
from jax.experimental import pallas as pl


def kernel(p_change, weight):
    raise NotImplementedError("write your pallas kernel here")


# speedup vs baseline: 2.3540x; 2.3540x over previous
import functools
import jax
import jax.numpy as jnp
from jax import lax
from jax.experimental import pallas as pl
from jax.experimental.pallas import tpu as pltpu
from jax.experimental.pallas import tpu_sc as plsc

NE = 1024
ED = 256
PTH = 0.8
B, T = 16, 2048
NC, NS, L = 2, 16, 16
NW = NC * NS
HALF = T // 2
CH = 128
NCHUNK = HALF // CH
CPR = ED // L


def _sc_body(p_hbm, w_hbm, out_hbm, part_hbm,
             p_row, idx1, idx2, pfrep, z1b, z2b, outb, accb, sem):
  c = lax.axis_index("c")
  s = lax.axis_index("s")
  wid = s * NC + c
  b = wid // 2
  half = wid % 2
  t0 = half * HALF
  row_base = b * T + t0

  pltpu.sync_copy(p_hbm.at[b], p_row)

  iota = lax.iota(jnp.int32, L)

  def _carry_body(i, acc):
    pv = p_row[pl.ds(i * L, L)]
    pos = i * L + iota
    sig = (pv >= PTH) & (pos > 0)
    return acc + jnp.where(sig, 1, 0).astype(jnp.int32)

  carry_vec = lax.fori_loop(0, half * (HALF // L), _carry_body,
                            jnp.zeros((L,), jnp.int32))
  cum0 = jnp.sum(carry_vec)

  def _chunk_body(ch, carry):
    cum, acc_s, acc_q = carry
    base = t0 + ch * CH

    def _idx_body(j, cum_in):
      pv = p_row[pl.ds(base + j * L, L)]
      pos = base + j * L + iota
      sig = (pv >= PTH) & (pos > 0)
      sigi = jnp.where(sig, 1, 0).astype(jnp.int32)
      loc = plsc.cumsum(sigi) + cum_in
      i1 = jnp.minimum(loc, NE - 1)
      i2 = jnp.clip(jnp.where(sig, i1 - 1, i1 + 1), 0, NE)
      pf = jnp.where(sig, pv, 1.0 - pv)
      idx1[pl.ds(j * L, L)] = i1
      idx2[pl.ds(j * L, L)] = i2
      scat_base = j * (L * L) + iota * L
      for k in range(L):
        plsc.store_scatter(pfrep, [scat_base + k], pf)
      return jnp.max(loc)

    cum = lax.fori_loop(0, CH // L, _idx_body, cum)

    pltpu.async_copy(w_hbm.at[idx1], z1b, sem).wait()
    pltpu.async_copy(w_hbm.at[idx2], z2b, sem).wait()

    def _blend_body(r, bl_carry):
      a_s, a_q = bl_carry
      pf = pfrep[pl.ds(r * L, L)]
      for cix in range(CPR):
        z1 = z1b[r, pl.ds(cix * L, L)]
        z2 = z2b[r, pl.ds(cix * L, L)]
        outb[r, pl.ds(cix * L, L)] = z2 + pf * (z1 - z2)
        a_s = a_s + z1
        a_q = a_q + z1 * z1
      return (a_s, a_q)

    acc_s, acc_q = lax.fori_loop(0, CH, _blend_body, (acc_s, acc_q))

    pltpu.sync_copy(outb, out_hbm.at[pl.ds(row_base + ch * CH, CH)])
    return (cum, acc_s, acc_q)

  zero = jnp.zeros((L,), jnp.float32)
  _, acc_s, acc_q = lax.fori_loop(0, NCHUNK, _chunk_body, (cum0, zero, zero))

  accb[pl.ds(0, L)] = acc_s
  accb[pl.ds(L, L)] = acc_q
  pltpu.sync_copy(accb, part_hbm.at[wid])


_vq3_sc = functools.partial(
    pl.kernel,
    out_type=(jax.ShapeDtypeStruct((B * T, ED), jnp.float32),
              jax.ShapeDtypeStruct((NW, 2 * L), jnp.float32)),
    mesh=plsc.VectorSubcoreMesh(core_axis_name="c", subcore_axis_name="s",
                                num_cores=NC, num_subcores=NS),
    compiler_params=pltpu.CompilerParams(needs_layout_passes=False),
    scratch_types=[
        pltpu.VMEM((T,), jnp.float32),
        pltpu.VMEM((CH,), jnp.int32),
        pltpu.VMEM((CH,), jnp.int32),
        pltpu.VMEM((CH * L,), jnp.float32),
        pltpu.VMEM((CH, ED), jnp.float32),
        pltpu.VMEM((CH, ED), jnp.float32),
        pltpu.VMEM((CH, ED), jnp.float32),
        pltpu.VMEM((2 * L,), jnp.float32),
        pltpu.SemaphoreType.DMA,
    ],
)(_sc_body)


def kernel(p_change, weight):
  z_flat, parts = _vq3_sc(p_change, weight)
  z_out = z_flat.reshape(B, T, ED)
  n = B * T * ED
  ssum = jnp.sum(parts[:, :L])
  qsum = jnp.sum(parts[:, L:])
  v = (qsum - ssum * ssum / n) / (n - 1)
  return (z_out, v)
